# baseline (speedup 1.0000x reference)
import jax
import jax.numpy as jnp
from jax import lax
from jax.experimental import pallas as pl
from jax.experimental.pallas import tpu as pltpu
from jax.experimental.pallas import tpu_sc as plsc

EMBED_DIM = 32
NUM_CORES = 2
NUM_SUBCORES = 16
NUM_WORKERS = NUM_CORES * NUM_SUBCORES
VOCAB = 1000000
EBLK = 128
NFULL = VOCAB // EBLK
ETAIL = VOCAB - NFULL * EBLK
N_TAIL_T4 = ETAIL * EMBED_DIM // 128

CHUNK = 1024
NBUF = 3


def _sc_transpose_body(
    embT_hbm, tail_hbm, t4_hbm, in0, in1, out0, out1, is0, is1, os0, os1
):
    wid = lax.axis_index("s") * NUM_CORES + lax.axis_index("c")
    iota = lax.iota(jnp.int32, 16)
    d_idx = [iota, iota + 16]

    @pl.when(wid == 0)
    def _tail():
        pltpu.sync_copy(tail_hbm, in0.at[pl.ds(0, N_TAIL_T4)])
        pltpu.sync_copy(
            in0.at[pl.ds(0, N_TAIL_T4)], t4_hbm.at[pl.ds(NFULL * 32, N_TAIL_T4)]
        )

    def transpose_block(in_v, out_v):
        @plsc.parallel_loop(0, EBLK, step=1, unroll=8)
        def _(e):
            e_splat = lax.broadcast(e, (16,))
            for half in range(2):
                vals = plsc.load_gather(in_v, [d_idx[half], e_splat])
                out_v[e // 4, pl.ds((e % 4) * 32 + 16 * half, 16)] = vals

    def do(b, in_v, out_v, isem, osem):
        pltpu.make_async_copy(
            embT_hbm.at[:, pl.ds(0, EBLK)], in_v, isem
        ).wait()
        transpose_block(in_v, out_v)
        pltpu.async_copy(out_v, t4_hbm.at[pl.ds(b * 32, 32)], osem)

    def step(kk, carry):
        b0 = wid + NUM_WORKERS * (2 * kk)
        b1 = wid + NUM_WORKERS * (2 * kk + 1)
        v0 = b0 < NFULL
        v1 = b1 < NFULL

        @pl.when(v0)
        def _i0():
            pltpu.async_copy(embT_hbm.at[:, pl.ds(b0 * EBLK, EBLK)], in0, is0)

        @pl.when(v1)
        def _i1():
            pltpu.async_copy(embT_hbm.at[:, pl.ds(b1 * EBLK, EBLK)], in1, is1)

        @pl.when(v0)
        def _c0():
            do(b0, in0, out0, is0, os0)

        @pl.when(v1)
        def _c1():
            do(b1, in1, out1, is1, os1)

        @pl.when(v0)
        def _w0():
            pltpu.make_async_copy(out0, t4_hbm.at[pl.ds(0, 32)], os0).wait()

        @pl.when(v1)
        def _w1():
            pltpu.make_async_copy(out1, t4_hbm.at[pl.ds(0, 32)], os1).wait()

        return carry

    n_pairs = (NFULL // NUM_WORKERS + 2) // 2
    lax.fori_loop(0, n_pairs, step, 0)


def _gather_body(idx_hbm, table_hbm, out_hbm, idx_v, rows_v, *sems):
    gsems, ssems = sems[:NBUF], sems[NBUF:]
    n_chunks = idx_hbm.shape[0] // (NUM_WORKERS * CHUNK)
    wid = lax.axis_index("s") * NUM_CORES + lax.axis_index("c")
    base = wid * (n_chunks * CHUNK)
    gathers = [None] * n_chunks
    stores = [None] * n_chunks

    def start_gather(c):
        b = c % NBUF
        pltpu.sync_copy(idx_hbm.at[pl.ds(base + c * CHUNK, CHUNK)], idx_v.at[b])
        gathers[c] = pltpu.async_copy(
            table_hbm.at[idx_v.at[b]], rows_v.at[b], gsems[b]
        )

    start_gather(0)
    for c in range(n_chunks):
        b = c % NBUF
        if c + 1 < n_chunks:
            if c + 1 >= NBUF:
                stores[c + 1 - NBUF].wait()
            start_gather(c + 1)
        gathers[c].wait()
        stores[c] = pltpu.async_copy(
            rows_v.at[b], out_hbm.at[pl.ds(base + c * CHUNK, CHUNK)], ssems[b]
        )
    for c in range(max(0, n_chunks - NBUF), n_chunks):
        stores[c].wait()


def _embed_lookup(idx_flat, table):
    n = idx_flat.shape[0]
    mesh = plsc.VectorSubcoreMesh(core_axis_name="c", subcore_axis_name="s")
    tail_lin = table[NFULL * EBLK :, :].reshape(N_TAIL_T4, 128)
    t4 = pl.kernel(
        _sc_transpose_body,
        out_type=jax.ShapeDtypeStruct((VOCAB * EMBED_DIM // 128, 128), jnp.float32),
        mesh=mesh,
        scratch_types=[
            pltpu.VMEM((32, EBLK), jnp.float32),
            pltpu.VMEM((32, EBLK), jnp.float32),
            pltpu.VMEM((32, EBLK), jnp.float32),
            pltpu.VMEM((32, EBLK), jnp.float32),
        ]
        + [pltpu.SemaphoreType.DMA] * 4,
        compiler_params=pltpu.CompilerParams(
            use_tc_tiling_on_sc=True, needs_layout_passes=False
        ),
    )(table.T, tail_lin)
    t_lin = t4.reshape(VOCAB, EMBED_DIM)
    return pl.kernel(
        _gather_body,
        out_type=jax.ShapeDtypeStruct((n, EMBED_DIM), jnp.float32),
        mesh=mesh,
        scratch_types=[
            pltpu.VMEM((NBUF, CHUNK), jnp.int32),
            pltpu.VMEM((NBUF, CHUNK, EMBED_DIM), jnp.float32),
        ]
        + [pltpu.SemaphoreType.DMA] * (2 * NBUF),
        compiler_params=pltpu.CompilerParams(use_tc_tiling_on_sc=False),
    )(idx_flat, t_lin)


def kernel(embedding_input, embedding):
    batch, hist = embedding_input.shape
    idx_flat = embedding_input.reshape(-1).astype(jnp.int32)
    out = _embed_lookup(idx_flat, embedding)
    return out.reshape(batch, hist, EMBED_DIM)
